# baseline (speedup 1.0000x reference)
import functools

import jax
import jax.numpy as jnp
from jax import lax
from jax.experimental import pallas as pl
from jax.experimental.pallas import tpu as pltpu
from jax.experimental.pallas import tpu_sc as plsc

_BATCH = 16384
_HIST = 200
_D = 64
_B = _BATCH * _HIST

_IB = 512
_GSUB = 128


@functools.cache
def _build():
    info = plsc.get_sparse_core_info()
    nc, ns = info.num_cores, info.num_subcores
    nw = nc * ns
    bpw = _B // nw
    nit = bpw // _IB
    mesh = plsc.VectorSubcoreMesh(core_axis_name="c", subcore_axis_name="s")

    def body(x_hbm, tab_hbm, out_hbm, idx_v, rows_v, sem_g):
        wid = lax.axis_index("s") * nc + lax.axis_index("c")
        base = wid * bpw

        @pl.loop(0, nit)
        def _(g):
            off = base + g * _IB
            pltpu.sync_copy(x_hbm.at[pl.ds(off, _IB)], idx_v)
            descs = [
                pltpu.async_copy(
                    tab_hbm.at[idx_v.at[pl.ds(j * _GSUB, _GSUB)]],
                    rows_v.at[pl.ds(j * _GSUB, _GSUB), :],
                    sem_g,
                )
                for j in range(_IB // _GSUB)
            ]
            for d in descs:
                d.wait()
            pltpu.sync_copy(rows_v, out_hbm.at[pl.ds(off, _IB)])

    return pl.kernel(
        body,
        out_type=jax.ShapeDtypeStruct((_B, _D), jnp.float32),
        mesh=mesh,
        scratch_types=[
            pltpu.VMEM((_IB,), jnp.int32),
            pltpu.VMEM((_IB, _D), jnp.float32),
            pltpu.SemaphoreType.DMA,
        ],
        compiler_params=pltpu.CompilerParams(use_tc_tiling_on_sc=False),
    )


def kernel(x, embeddings):
    run = _build()
    xf = x.reshape(-1).astype(jnp.int32)
    out = run(xf, embeddings)
    return out.reshape(_BATCH, _HIST, _D)


# speedup vs baseline: 1.0705x; 1.0705x over previous
import functools

import jax
import jax.numpy as jnp
from jax import lax
from jax.experimental import pallas as pl
from jax.experimental.pallas import tpu as pltpu
from jax.experimental.pallas import tpu_sc as plsc

_BATCH = 16384
_HIST = 200
_D = 64
_B = _BATCH * _HIST

_IB = 512
_GSUB = 128
_NSUB = _IB // _GSUB


@functools.cache
def _build():
    info = plsc.get_sparse_core_info()
    nc, ns = info.num_cores, info.num_subcores
    nw = nc * ns
    bpw = _B // nw
    nit = bpw // _IB
    mesh = plsc.VectorSubcoreMesh(core_axis_name="c", subcore_axis_name="s")

    def body(x_hbm, tab_hbm, out_hbm, idx0, idx1, rows0, rows1,
             si0, si1, sg0, sg1, so0, so1):
        idx = (idx0, idx1)
        rows = (rows0, rows1)
        si = (si0, si1)
        sg = (sg0, sg1)
        so = (so0, so1)
        wid = lax.axis_index("s") * nc + lax.axis_index("c")
        base = wid * bpw

        def fire_idx(chunk, b):
            pltpu.async_copy(
                x_hbm.at[pl.ds(base + chunk * _IB, _IB)], idx[b], si[b])

        def step(chunk, b, wait_out, prefetch):
            off = base + chunk * _IB
            out_slc = out_hbm.at[pl.ds(off, _IB)]
            if wait_out:
                pltpu.make_async_copy(rows[b], out_slc, so[b]).wait()
            pltpu.make_async_copy(
                x_hbm.at[pl.ds(off, _IB)], idx[b], si[b]).wait()
            descs = [
                pltpu.async_copy(
                    tab_hbm.at[idx[b].at[pl.ds(j * _GSUB, _GSUB)]],
                    rows[b].at[pl.ds(j * _GSUB, _GSUB), :],
                    sg[b],
                )
                for j in range(_NSUB)
            ]
            for d in descs:
                d.wait()
            pltpu.async_copy(rows[b], out_slc, so[b])
            if prefetch:
                fire_idx(chunk + 2, b)

        fire_idx(0, 0)
        fire_idx(1, 1)
        step(0, 0, wait_out=False, prefetch=True)
        step(1, 1, wait_out=False, prefetch=True)

        @pl.loop(2, nit - 2, step=2)
        def _(g):
            step(g, 0, wait_out=True, prefetch=True)
            step(g + 1, 1, wait_out=True, prefetch=True)

        step(nit - 2, 0, wait_out=True, prefetch=False)
        step(nit - 1, 1, wait_out=True, prefetch=False)
        pltpu.make_async_copy(rows0, out_hbm.at[pl.ds(base, _IB)], so0).wait()
        pltpu.make_async_copy(rows1, out_hbm.at[pl.ds(base, _IB)], so1).wait()

    return pl.kernel(
        body,
        out_type=jax.ShapeDtypeStruct((_B, _D), jnp.float32),
        mesh=mesh,
        scratch_types=[
            pltpu.VMEM((_IB,), jnp.int32),
            pltpu.VMEM((_IB,), jnp.int32),
            pltpu.VMEM((_IB, _D), jnp.float32),
            pltpu.VMEM((_IB, _D), jnp.float32),
            pltpu.SemaphoreType.DMA,
            pltpu.SemaphoreType.DMA,
            pltpu.SemaphoreType.DMA,
            pltpu.SemaphoreType.DMA,
            pltpu.SemaphoreType.DMA,
            pltpu.SemaphoreType.DMA,
        ],
        compiler_params=pltpu.CompilerParams(use_tc_tiling_on_sc=False),
    )


def kernel(x, embeddings):
    run = _build()
    xf = x.reshape(-1).astype(jnp.int32)
    out = run(xf, embeddings)
    return out.reshape(_BATCH, _HIST, _D)


# speedup vs baseline: 1.0718x; 1.0012x over previous
import functools

import jax
import jax.numpy as jnp
from jax import lax
from jax.experimental import pallas as pl
from jax.experimental.pallas import tpu as pltpu
from jax.experimental.pallas import tpu_sc as plsc

_BATCH = 16384
_HIST = 200
_D = 64

_CB = 4
_ROWS = _CB * _HIST
_GS = (128, 72)


@functools.cache
def _build():
    info = plsc.get_sparse_core_info()
    nc, ns = info.num_cores, info.num_subcores
    nw = nc * ns
    bpw = _BATCH // nw
    nit = bpw // _CB
    mesh = plsc.VectorSubcoreMesh(core_axis_name="c", subcore_axis_name="s")

    def body(x_hbm, tab_hbm, out_hbm, idx0, idx1, rows0, rows1,
             si0, si1, sg0, sg1, so0, so1):
        idx = (idx0, idx1)
        rows = (rows0, rows1)
        si = (si0, si1)
        sg = (sg0, sg1)
        so = (so0, so1)
        wid = lax.axis_index("s") * nc + lax.axis_index("c")
        base = wid * bpw

        def fire_idx(chunk, b):
            pltpu.async_copy(
                x_hbm.at[pl.ds(base + chunk * _CB, _CB), :], idx[b], si[b])

        def step(chunk, b, wait_out, prefetch):
            b0 = base + chunk * _CB
            out_slc = out_hbm.at[pl.ds(b0, _CB)]
            if wait_out:
                pltpu.make_async_copy(rows[b], out_slc, so[b]).wait()
            pltpu.make_async_copy(
                x_hbm.at[pl.ds(b0, _CB), :], idx[b], si[b]).wait()
            descs = []
            for r in range(_CB):
                col = 0
                for g in _GS:
                    descs.append(pltpu.async_copy(
                        tab_hbm.at[idx[b].at[r, pl.ds(col, g)]],
                        rows[b].at[r, pl.ds(col, g), :],
                        sg[b],
                    ))
                    col += g
            for d in descs:
                d.wait()
            pltpu.async_copy(rows[b], out_slc, so[b])
            if prefetch:
                fire_idx(chunk + 2, b)

        fire_idx(0, 0)
        fire_idx(1, 1)
        step(0, 0, wait_out=False, prefetch=True)
        step(1, 1, wait_out=False, prefetch=True)

        @pl.loop(2, nit - 2, step=2)
        def _(g):
            step(g, 0, wait_out=True, prefetch=True)
            step(g + 1, 1, wait_out=True, prefetch=True)

        step(nit - 2, 0, wait_out=True, prefetch=False)
        step(nit - 1, 1, wait_out=True, prefetch=False)
        out0 = out_hbm.at[pl.ds(base, _CB)]
        pltpu.make_async_copy(rows0, out0, so0).wait()
        pltpu.make_async_copy(rows1, out0, so1).wait()

    return pl.kernel(
        body,
        out_type=jax.ShapeDtypeStruct((_BATCH, _HIST, _D), jnp.float32),
        mesh=mesh,
        scratch_types=[
            pltpu.VMEM((_CB, _HIST), jnp.int32),
            pltpu.VMEM((_CB, _HIST), jnp.int32),
            pltpu.VMEM((_CB, _HIST, _D), jnp.float32),
            pltpu.VMEM((_CB, _HIST, _D), jnp.float32),
            pltpu.SemaphoreType.DMA,
            pltpu.SemaphoreType.DMA,
            pltpu.SemaphoreType.DMA,
            pltpu.SemaphoreType.DMA,
            pltpu.SemaphoreType.DMA,
            pltpu.SemaphoreType.DMA,
        ],
        compiler_params=pltpu.CompilerParams(use_tc_tiling_on_sc=False),
    )


def kernel(x, embeddings):
    run = _build()
    return run(x.astype(jnp.int32), embeddings)
